# baseline (speedup 1.0000x reference)
import jax
import jax.numpy as jnp
from jax.experimental import pallas as pl
from jax.experimental.pallas import tpu as pltpu

_BN = 1024
_NBUF = 4


def _make_outer(b, v, d):
    def outer(embs_t_ref, ids_hbm_ref, out_hbm_ref):
        e = embs_t_ref[...]
        norm = jnp.sqrt(jnp.sum(e * e, axis=0, keepdims=True))
        normed = e / jnp.maximum(norm, 1e-12)

        def inner(ids_blk_ref, out_blk_ref):
            out_blk_ref[...] = jnp.dot(
                normed, ids_blk_ref[...], preferred_element_type=jnp.float32
            )

        pipe = pltpu.emit_pipeline(
            inner,
            grid=(b // _BN,),
            in_specs=[
                pl.BlockSpec(
                    (v, _BN),
                    lambda i: (0, i),
                    pipeline_mode=pl.Buffered(buffer_count=_NBUF),
                )
            ],
            out_specs=[pl.BlockSpec((d, _BN), lambda i: (0, i))],
        )
        pipe(ids_hbm_ref, out_hbm_ref)

    return outer


def kernel(ids, embs):
    b, v = ids.shape
    _, d = embs.shape
    ids_t = ids.T
    embs_t = embs.T
    out_t = pl.pallas_call(
        _make_outer(b, v, d),
        in_specs=[
            pl.BlockSpec((d, v), lambda: (0, 0)),
            pl.BlockSpec(memory_space=pl.ANY),
        ],
        out_specs=pl.BlockSpec(memory_space=pl.ANY),
        out_shape=jax.ShapeDtypeStruct((d, b), jnp.float32),
    )(embs_t, ids_t)
    return out_t.T


# speedup vs baseline: 1.0726x; 1.0726x over previous
import jax
import jax.numpy as jnp
from jax.experimental import pallas as pl
from jax.experimental.pallas import tpu as pltpu

_BN = 2048


def _embed_kernel(embs_t_ref, ids_t_ref, out_ref):
    e = embs_t_ref[...]
    norm = jnp.sqrt(jnp.sum(e * e, axis=0, keepdims=True))
    normed = e / jnp.maximum(norm, 1e-12)
    out_ref[...] = jnp.dot(
        normed, ids_t_ref[...], preferred_element_type=jnp.float32
    )


def kernel(ids, embs):
    b, v = ids.shape
    _, d = embs.shape
    ids_t = ids.T
    embs_t = embs.T
    out_t = pl.pallas_call(
        _embed_kernel,
        grid=(b // _BN,),
        in_specs=[
            pl.BlockSpec((d, v), lambda i: (0, 0)),
            pl.BlockSpec((v, _BN), lambda i: (0, i)),
        ],
        out_specs=pl.BlockSpec((d, _BN), lambda i: (0, i)),
        out_shape=jax.ShapeDtypeStruct((d, b), jnp.float32),
        compiler_params=pltpu.CompilerParams(
            dimension_semantics=("parallel",)
        ),
    )(embs_t, ids_t)
    return out_t.T
